# baseline (speedup 1.0000x reference)
import functools

import jax
import jax.numpy as jnp
from jax.experimental import pallas as pl
from jax.experimental.pallas import tpu as pltpu


def _fused_kernel(x_ref, w2_ref, w1_ref, wsc_ref, o_ref, xpad_ref, o2pad_ref,
                  *, h, w):
    c_in = x_ref.shape[1]
    hw = h * w
    cmid = w2_ref.shape[2]
    cout = w1_ref.shape[2]
    f32 = jnp.float32
    bf16 = jnp.bfloat16

    xr = jnp.maximum(x_ref[0], 0.0).astype(bf16).reshape(c_in, hw)
    xrT = jnp.transpose(xr)

    xpad_ref[...] = jnp.zeros(xpad_ref.shape, bf16)
    xpad_ref[1:h + 1, 1:w + 1, :] = xrT.reshape(h, w, c_in)
    xp = xpad_ref[...]
    xsh = [xp[:, dw:dw + w, :] for dw in range(3)]

    acc2 = jnp.zeros((hw, cmid), f32)
    for dh in range(3):
        for dw in range(3):
            patch = xsh[dw][dh:dh + h].reshape(hw, c_in)
            acc2 = acc2 + jnp.dot(patch, w2_ref[3 * dh + dw],
                                  preferred_element_type=f32)
    out2 = jnp.maximum(acc2, 0.0).astype(bf16)

    o2pad_ref[...] = jnp.zeros(o2pad_ref.shape, bf16)
    o2pad_ref[0:h, 0:w, :] = out2.reshape(h, w, cmid)
    op = o2pad_ref[...]
    osh = [op[:, dw:dw + w, :] for dw in range(2)]

    def tap(t, dh, dw):
        patch = osh[dw][dh:dh + h].reshape(hw, cmid)
        return jnp.dot(patch, w1_ref[t], preferred_element_type=f32)

    p00 = tap(0, 0, 0) + jnp.dot(xrT, wsc_ref[...], preferred_element_type=f32)
    p01 = tap(1, 0, 0) + tap(2, 0, 1)
    p10 = tap(3, 0, 0) + tap(4, 1, 0)
    p11 = tap(5, 0, 0) + tap(6, 0, 1) + tap(7, 1, 0) + tap(8, 1, 1)

    big = jnp.stack([p00.astype(bf16), p01.astype(bf16),
                     p10.astype(bf16), p11.astype(bf16)], axis=0)
    big = big.reshape(2, 2, h, w, cout)
    big = jnp.transpose(big, (2, 0, 3, 1, 4))
    big = big.reshape(4 * hw, cout)
    t = jnp.transpose(big).astype(o_ref.dtype)
    o_ref[0] = t.reshape(cout, 2 * h, 2 * w)


def kernel(x, w_d2, w_d1, w_sc):
    n, cin, h, w = x.shape
    cmid = w_d2.shape[1]
    cout = w_d1.shape[1]
    bf16 = jnp.bfloat16

    w2f = jnp.transpose(w_d2, (2, 3, 0, 1))[::-1, ::-1].astype(bf16)
    w2t = w2f.reshape(9, cin, cmid)
    w1f = jnp.transpose(w_d1, (2, 3, 0, 1))[::-1, ::-1].astype(bf16)
    sub_kk = ((1, 1), (1, 0), (1, 2), (0, 1), (2, 1),
              (0, 0), (0, 2), (2, 0), (2, 2))
    w1t = jnp.stack([w1f[kh, kw] for kh, kw in sub_kk], axis=0)
    wsc = w_sc[:, :, 0, 0].astype(bf16)

    out = pl.pallas_call(
        functools.partial(_fused_kernel, h=h, w=w),
        out_shape=jax.ShapeDtypeStruct((n, cout, 2 * h, 2 * w), x.dtype),
        grid=(n,),
        in_specs=[
            pl.BlockSpec((1, cin, h, w), lambda b: (b, 0, 0, 0)),
            pl.BlockSpec((9, cin, cmid), lambda b: (0, 0, 0)),
            pl.BlockSpec((9, cmid, cout), lambda b: (0, 0, 0)),
            pl.BlockSpec((cin, cout), lambda b: (0, 0)),
        ],
        out_specs=pl.BlockSpec((1, cout, 2 * h, 2 * w), lambda b: (b, 0, 0, 0)),
        scratch_shapes=[
            pltpu.VMEM((h + 2, w + 2, cin), bf16),
            pltpu.VMEM((h + 1, w + 1, cmid), bf16),
        ],
        compiler_params=pltpu.CompilerParams(
            dimension_semantics=("parallel",),
            vmem_limit_bytes=48 * 2 ** 20,
        ),
    )(x, w2t, w1t, wsc)

    return out


# speedup vs baseline: 1.9707x; 1.9707x over previous
import functools

import jax
import jax.numpy as jnp
from jax.experimental import pallas as pl
from jax.experimental.pallas import tpu as pltpu


def _fused_kernel(x_ref, w2_ref, w1_ref, wsc_ref, o_ref, xpad_ref, o2pad_ref,
                  *, h, w):
    c_in = x_ref.shape[1]
    hw = h * w
    cmid = w2_ref.shape[2]
    cout = w1_ref.shape[2]
    f32 = jnp.float32
    bf16 = jnp.bfloat16

    xrT = jnp.transpose(jnp.maximum(x_ref[0], 0.0).astype(bf16))

    xpad_ref[...] = jnp.zeros(xpad_ref.shape, bf16)
    xpad_ref[1:h + 1, 1:w + 1, :] = xrT.reshape(h, w, c_in)
    xp = xpad_ref[...]
    xsh = [xp[:, dw:dw + w, :] for dw in range(3)]

    acc2 = jnp.zeros((hw, cmid), f32)
    for dh in range(3):
        for dw in range(3):
            patch = xsh[dw][dh:dh + h].reshape(hw, c_in)
            acc2 = acc2 + jnp.dot(patch, w2_ref[3 * dh + dw],
                                  preferred_element_type=f32)
    out2 = jnp.maximum(acc2, 0.0).astype(bf16)

    o2pad_ref[...] = jnp.zeros(o2pad_ref.shape, bf16)
    o2pad_ref[0:h, 0:w, :] = out2.reshape(h, w, cmid)
    op = o2pad_ref[...]
    osh = [op[:, dw:dw + w, :] for dw in range(2)]

    def tap(t, dh, dw):
        patch = osh[dw][dh:dh + h].reshape(hw, cmid)
        return jnp.dot(patch, w1_ref[t], preferred_element_type=f32)

    p00 = tap(0, 0, 0) + jnp.dot(xrT, wsc_ref[...], preferred_element_type=f32)
    o_ref[0, 0] = p00.reshape(h, w, cout).astype(o_ref.dtype)
    p01 = tap(1, 0, 0) + tap(2, 0, 1)
    o_ref[0, 1] = p01.reshape(h, w, cout).astype(o_ref.dtype)
    p10 = tap(3, 0, 0) + tap(4, 1, 0)
    o_ref[0, 2] = p10.reshape(h, w, cout).astype(o_ref.dtype)
    p11 = tap(5, 0, 0) + tap(6, 0, 1) + tap(7, 1, 0) + tap(8, 1, 1)
    o_ref[0, 3] = p11.reshape(h, w, cout).astype(o_ref.dtype)


def kernel(x, w_d2, w_d1, w_sc):
    n, cin, h, w = x.shape
    cmid = w_d2.shape[1]
    cout = w_d1.shape[1]
    bf16 = jnp.bfloat16

    x2 = x.reshape(n, cin, h * w)

    w2f = jnp.transpose(w_d2, (2, 3, 0, 1))[::-1, ::-1].astype(bf16)
    w2t = w2f.reshape(9, cin, cmid)
    w1f = jnp.transpose(w_d1, (2, 3, 0, 1))[::-1, ::-1].astype(bf16)
    sub_kk = ((1, 1), (1, 0), (1, 2), (0, 1), (2, 1),
              (0, 0), (0, 2), (2, 0), (2, 2))
    w1t = jnp.stack([w1f[kh, kw] for kh, kw in sub_kk], axis=0)
    wsc = w_sc[:, :, 0, 0].astype(bf16)

    out = pl.pallas_call(
        functools.partial(_fused_kernel, h=h, w=w),
        out_shape=jax.ShapeDtypeStruct((n, 4, h, w, cout), x.dtype),
        grid=(n,),
        in_specs=[
            pl.BlockSpec((1, cin, h * w), lambda b: (b, 0, 0)),
            pl.BlockSpec((9, cin, cmid), lambda b: (0, 0, 0)),
            pl.BlockSpec((9, cmid, cout), lambda b: (0, 0, 0)),
            pl.BlockSpec((cin, cout), lambda b: (0, 0)),
        ],
        out_specs=pl.BlockSpec((1, 4, h, w, cout), lambda b: (b, 0, 0, 0, 0)),
        scratch_shapes=[
            pltpu.VMEM((h + 2, w + 2, cin), bf16),
            pltpu.VMEM((h + 1, w + 1, cmid), bf16),
        ],
        compiler_params=pltpu.CompilerParams(
            dimension_semantics=("parallel",),
            vmem_limit_bytes=48 * 2 ** 20,
        ),
    )(x2, w2t, w1t, wsc)

    out = out.reshape(n, 2, 2, h, w, cout)
    out = jnp.transpose(out, (0, 5, 3, 1, 4, 2)).reshape(n, cout, 2 * h, 2 * w)
    return out
